# Initial kernel scaffold; baseline (speedup 1.0000x reference)
import jax
import jax.numpy as jnp
from jax.experimental import pallas as pl


def kernel(x, table, bias1, W2, b2, Wcp, bcp, Wwdl, bwdl):
    raise NotImplementedError("write your pallas kernel here")


# speedup vs baseline: 1.4658x; 1.4658x over previous
import functools

import jax
import jax.numpy as jnp
from jax import lax
from jax.experimental import pallas as pl
from jax.experimental.pallas import tpu as pltpu
from jax.experimental.pallas import tpu_sc as plsc

B = 16384
L = 32
H = 512
NC = 2
NS = 16
NW = NC * NS
BPW = B // NW
G = 4
GROUPS = BPW // G


def _bag_body(x_hbm, table_hbm, out_hbm, idx_v, rows_v, acc_v, sem):
    wid = lax.axis_index("s") * NC + lax.axis_index("c")
    base = wid * BPW

    def group_body(g, _):
        s0 = base + g * G
        pltpu.sync_copy(x_hbm.at[pl.ds(s0 * L, G * L)], idx_v)
        pltpu.async_copy(table_hbm.at[idx_v], rows_v, sem).wait()

        def col_body(c, _2):
            co = c * 16
            for s in range(G):
                def l_body(l8, acc):
                    r = s * L + l8 * 8
                    v = [rows_v[r + j, pl.ds(co, 16)] for j in range(8)]
                    t = ((v[0] + v[1]) + (v[2] + v[3])) + (
                        (v[4] + v[5]) + (v[6] + v[7]))
                    return acc + t

                acc = lax.fori_loop(0, L // 8, l_body,
                                    jnp.zeros((16,), jnp.float32))
                acc_v[pl.ds(s * H + co, 16)] = acc
            return 0

        lax.fori_loop(0, H // 16, col_body, 0)
        pltpu.sync_copy(acc_v, out_hbm.at[pl.ds(s0 * H, G * H)])
        return 0

    lax.fori_loop(0, GROUPS, group_body, 0)


@functools.partial(
    pl.kernel,
    mesh=plsc.VectorSubcoreMesh(core_axis_name="c", subcore_axis_name="s"),
    out_type=jax.ShapeDtypeStruct((B * H,), jnp.float32),
    scratch_types=[
        pltpu.VMEM((G * L,), jnp.int32),
        pltpu.VMEM((G * L, H), jnp.float32),
        pltpu.VMEM((G * H,), jnp.float32),
        pltpu.SemaphoreType.DMA,
    ],
)
def _bag_kernel(x_hbm, table_hbm, out_hbm, idx_v, rows_v, acc_v, sem):
    _bag_body(x_hbm, table_hbm, out_hbm, idx_v, rows_v, acc_v, sem)


BB = 1024


def _head_body(bag_ref, b1_ref, w2t_ref, b2_ref, wht_ref, bh_ref, out_ref):
    h = jnp.maximum(bag_ref[...] + b1_ref[...], 0.0)
    h2 = jnp.dot(h, w2t_ref[...], preferred_element_type=jnp.float32)
    h2 = jnp.maximum(h2 + b2_ref[...], 0.0)
    out_ref[...] = (
        jnp.dot(h2, wht_ref[...], preferred_element_type=jnp.float32)
        + bh_ref[...]
    )


def _head_call(bag, b1, w2t, b2, wht, bh):
    return pl.pallas_call(
        _head_body,
        grid=(B // BB,),
        in_specs=[
            pl.BlockSpec((BB, H), lambda i: (i, 0)),
            pl.BlockSpec((H,), lambda i: (0,)),
            pl.BlockSpec((H, 32), lambda i: (0, 0)),
            pl.BlockSpec((32,), lambda i: (0,)),
            pl.BlockSpec((32, 4), lambda i: (0, 0)),
            pl.BlockSpec((4,), lambda i: (0,)),
        ],
        out_specs=pl.BlockSpec((BB, 4), lambda i: (i, 0)),
        out_shape=jax.ShapeDtypeStruct((B, 4), jnp.float32),
    )(bag, b1, w2t, b2, wht, bh)


def kernel(x, table, bias1, W2, b2, Wcp, bcp, Wwdl, bwdl):
    xf = x.reshape(-1)
    bag = _bag_kernel(xf, table).reshape(B, H)
    wht = jnp.concatenate([Wcp, Wwdl], axis=0).T
    bh = jnp.concatenate([bcp, bwdl], axis=0)
    out = _head_call(bag, bias1, W2.T, b2, wht, bh)
    return out[:, :1], out[:, 1:4]


# speedup vs baseline: 2.5167x; 1.7170x over previous
import functools

import jax
import jax.numpy as jnp
from jax import lax
from jax.experimental import pallas as pl
from jax.experimental.pallas import tpu as pltpu
from jax.experimental.pallas import tpu_sc as plsc

B = 16384
L = 32
H = 512
NC = 2
NS = 16
NW = NC * NS
BPW = B // NW
GB = 2
RG = GB * L
GROUPS = BPW // GB
IDX_ROWS = BPW * L // RG


def _reduce_group(rows_v, acc_v):
    def col_body(c, _2):
        co = c * 16
        for s in range(GB):
            def l_body(l8, acc):
                r = s * L + l8 * 8
                v = [rows_v[r + j, pl.ds(co, 16)] for j in range(8)]
                t = ((v[0] + v[1]) + (v[2] + v[3])) + (
                    (v[4] + v[5]) + (v[6] + v[7]))
                return acc + t

            acc = lax.fori_loop(0, L // 8, l_body,
                                jnp.zeros((16,), jnp.float32))
            acc_v[pl.ds(s * H + co, 16)] = acc
        return 0

    lax.fori_loop(0, H // 16, col_body, 0)


def _bag_body(x_hbm, table_hbm, out_hbm, idx_v, rows0, rows1, acc0, acc1,
              gsem0, gsem1, osem0, osem1):
    wid = lax.axis_index("s") * NC + lax.axis_index("c")
    base = wid * BPW
    rows = (rows0, rows1)
    accs = (acc0, acc1)
    gsems = (gsem0, gsem1)
    osems = (osem0, osem1)

    pltpu.sync_copy(x_hbm.at[pl.ds(wid * IDX_ROWS, IDX_ROWS)], idx_v)

    pltpu.async_copy(table_hbm.at[idx_v.at[0]], rows0, gsem0)
    pltpu.async_copy(table_hbm.at[idx_v.at[1]], rows1, gsem1)

    def pair_body(p, _):
        for b in range(2):
            g = p * 2 + b
            pltpu.make_async_copy(table_hbm.at[idx_v.at[g]], rows[b],
                                  gsems[b]).wait()
            @pl.when(p > 0)
            def _wait_out():
                pltpu.make_async_copy(
                    accs[b],
                    out_hbm.at[pl.ds((base + (g - 2) * GB) * H, GB * H)],
                    osems[b]).wait()

            _reduce_group(rows[b], accs[b])

            @pl.when(g + 2 < GROUPS)
            def _next_gather():
                pltpu.async_copy(table_hbm.at[idx_v.at[g + 2]], rows[b],
                                 gsems[b])

            pltpu.async_copy(
                accs[b],
                out_hbm.at[pl.ds((base + g * GB) * H, GB * H)],
                osems[b])
        return 0

    lax.fori_loop(0, GROUPS // 2, pair_body, 0)

    for b in range(2):
        g = GROUPS - 2 + b
        pltpu.make_async_copy(
            accs[b],
            out_hbm.at[pl.ds((base + g * GB) * H, GB * H)],
            osems[b]).wait()


@functools.partial(
    pl.kernel,
    mesh=plsc.VectorSubcoreMesh(core_axis_name="c", subcore_axis_name="s"),
    out_type=jax.ShapeDtypeStruct((B * H,), jnp.float32),
    scratch_types=[
        pltpu.VMEM((IDX_ROWS, RG), jnp.int32),
        pltpu.VMEM((RG, H), jnp.float32),
        pltpu.VMEM((RG, H), jnp.float32),
        pltpu.VMEM((GB * H,), jnp.float32),
        pltpu.VMEM((GB * H,), jnp.float32),
        pltpu.SemaphoreType.DMA,
        pltpu.SemaphoreType.DMA,
        pltpu.SemaphoreType.DMA,
        pltpu.SemaphoreType.DMA,
    ],
)
def _bag_kernel(x_hbm, table_hbm, out_hbm, idx_v, rows0, rows1, acc0, acc1,
                gsem0, gsem1, osem0, osem1):
    _bag_body(x_hbm, table_hbm, out_hbm, idx_v, rows0, rows1, acc0, acc1,
              gsem0, gsem1, osem0, osem1)


BB = 1024


def _head_body(bag_ref, b1_ref, w2t_ref, b2_ref, wht_ref, bh_ref, out_ref):
    h = jnp.maximum(bag_ref[...] + b1_ref[...], 0.0)
    h2 = jnp.dot(h, w2t_ref[...], preferred_element_type=jnp.float32)
    h2 = jnp.maximum(h2 + b2_ref[...], 0.0)
    out_ref[...] = (
        jnp.dot(h2, wht_ref[...], preferred_element_type=jnp.float32)
        + bh_ref[...]
    )


def _head_call(bag, b1, w2t, b2, wht, bh):
    return pl.pallas_call(
        _head_body,
        grid=(B // BB,),
        in_specs=[
            pl.BlockSpec((BB, H), lambda i: (i, 0)),
            pl.BlockSpec((H,), lambda i: (0,)),
            pl.BlockSpec((H, 32), lambda i: (0, 0)),
            pl.BlockSpec((32,), lambda i: (0,)),
            pl.BlockSpec((32, 4), lambda i: (0, 0)),
            pl.BlockSpec((4,), lambda i: (0,)),
        ],
        out_specs=pl.BlockSpec((BB, 4), lambda i: (i, 0)),
        out_shape=jax.ShapeDtypeStruct((B, 4), jnp.float32),
    )(bag, b1, w2t, b2, wht, bh)


def kernel(x, table, bias1, W2, b2, Wcp, bcp, Wwdl, bwdl):
    x_idx = x.reshape(B * L // RG, RG)
    bag = _bag_kernel(x_idx, table).reshape(B, H)
    wht = jnp.concatenate([Wcp, Wwdl], axis=0).T
    bh = jnp.concatenate([bcp, bwdl], axis=0)
    out = _head_call(bag, bias1, W2.T, b2, wht, bh)
    return out[:, :1], out[:, 1:4]
